# baseline (speedup 1.0000x reference)
import functools

import jax
import jax.numpy as jnp
from jax import lax
from jax.experimental import pallas as pl
from jax.experimental.pallas import tpu as pltpu
from jax.experimental.pallas import tpu_sc as plsc

D = 64
V = 100000
HIST = 168
PRED = 24
NF_HIST = 7
NF_FUT = 3
LANES = 128
QROWS = 256


def _gather_all(tab, idx, b, nw):
    qb = b // QROWS
    hq = HIST * NF_HIST * qb // nw
    fq = PRED * NF_FUT * qb // nw
    s_per_w = b // nw

    mesh = plsc.VectorSubcoreMesh(core_axis_name="c", subcore_axis_name="s")
    nc = mesh.num_cores

    @functools.partial(
        pl.kernel,
        out_type=[
            jax.ShapeDtypeStruct((HIST, NF_HIST, b, D), jnp.float32),
            jax.ShapeDtypeStruct((PRED, NF_FUT, b, D), jnp.float32),
            jax.ShapeDtypeStruct((b, D), jnp.float32),
        ],
        mesh=mesh,
        compiler_params=pltpu.CompilerParams(use_tc_tiling_on_sc=False),
        scratch_types=[
            pltpu.VMEM((QROWS,), jnp.int32),
            pltpu.VMEM((QROWS, D), jnp.float32),
            pltpu.VMEM((s_per_w,), jnp.int32),
            pltpu.VMEM((s_per_w, D), jnp.float32),
            pltpu.SemaphoreType.DMA,
        ],
    )
    def k(tab_hbm, idx_hbm, oh_hbm, of_hbm, os_hbm,
          idx_v, rows_v, sidx_v, srows_v, sem):
        wid = lax.axis_index("s") * nc + lax.axis_index("c")

        def phase(out_hbm, n_chunks, nf, t0):
            per_t = nf * qb

            def chunk_body(c, carry):
                q = wid * n_chunks + c
                t = q // per_t
                rem = q % per_t
                f = 1 + rem // qb
                quarter = rem % qb
                src = ((t0 + t) * 8 + f) * b + quarter * QROWS
                pltpu.sync_copy(idx_hbm.at[pl.ds(src, QROWS)], idx_v)
                cps = [
                    pltpu.async_copy(
                        tab_hbm.at[idx_v.at[pl.ds(j * LANES, LANES)]],
                        rows_v.at[pl.ds(j * LANES, LANES)],
                        sem,
                    )
                    for j in range(QROWS // LANES)
                ]
                for cp in cps:
                    cp.wait()
                pltpu.sync_copy(
                    rows_v,
                    out_hbm.at[t, f - 1, pl.ds(quarter * QROWS, QROWS), :])
                return carry

            lax.fori_loop(0, n_chunks, chunk_body, 0)

        phase(oh_hbm, hq, NF_HIST, 0)
        phase(of_hbm, fq, NF_FUT, HIST)

        pltpu.sync_copy(idx_hbm.at[pl.ds(wid * s_per_w, s_per_w)], sidx_v)
        pltpu.async_copy(tab_hbm.at[sidx_v], srows_v, sem).wait()
        pltpu.sync_copy(srows_v, os_hbm.at[pl.ds(wid * s_per_w, s_per_w)])

    return k(tab, idx)


def kernel(inputs, tables):
    B, W, NI = inputs.shape
    total_window = HIST + PRED
    if W > total_window:
        inputs = inputs[:, -total_window:, :]
        W = total_window
    inputs = inputs.astype(jnp.int32)
    tab = tables.reshape(NI * V, D)

    tr = inputs.transpose(1, 2, 0)
    offs = (jnp.arange(NI, dtype=jnp.int32) * V)[None, :, None]
    idx = (tr + offs).reshape(-1)

    info = plsc.get_sparse_core_info()
    nw = info.num_cores * info.num_subcores

    oh, of, os_ = _gather_all(tab, idx, B, nw)
    static = os_.reshape(B, 1, D)
    historical = oh.transpose(2, 0, 1, 3)
    future = of.transpose(2, 0, 1, 3)
    return (static, historical, future)


# speedup vs baseline: 1.1283x; 1.1283x over previous
import functools

import jax
import jax.numpy as jnp
from jax import lax
from jax.experimental import pallas as pl
from jax.experimental.pallas import tpu as pltpu
from jax.experimental.pallas import tpu_sc as plsc

D = 64
V = 100000
HIST = 168
PRED = 24
NF_HIST = 7
NF_FUT = 3
LANES = 128


def _gather_all(tab, idx3, b, nw):
    bg = b // LANES
    hc = HIST * bg // nw
    fc = PRED * bg // nw
    s_per_w = b // nw

    mesh = plsc.VectorSubcoreMesh(core_axis_name="c", subcore_axis_name="s")
    nc = mesh.num_cores

    @functools.partial(
        pl.kernel,
        out_type=[
            jax.ShapeDtypeStruct((HIST, NF_HIST, b, D), jnp.float32),
            jax.ShapeDtypeStruct((PRED, NF_FUT, b, D), jnp.float32),
            jax.ShapeDtypeStruct((b, D), jnp.float32),
        ],
        mesh=mesh,
        compiler_params=pltpu.CompilerParams(use_tc_tiling_on_sc=False),
        scratch_types=[
            pltpu.VMEM((2, 8, LANES), jnp.int32),
            pltpu.VMEM((2, NF_HIST * LANES, D), jnp.float32),
            pltpu.VMEM((s_per_w,), jnp.int32),
            pltpu.VMEM((s_per_w, D), jnp.float32),
            pltpu.SemaphoreType.DMA,
            pltpu.SemaphoreType.DMA,
            pltpu.SemaphoreType.DMA,
        ],
    )
    def k(tab_hbm, idx_hbm, oh_hbm, of_hbm, os_hbm,
          idx_v, rows_v, sidx_v, srows_v, gsem, osem0, osem1):
        wid = lax.axis_index("s") * nc + lax.axis_index("c")
        osems = [osem0, osem1]

        def phase(out_hbm, n_chunks, nf, t0):
            def do_chunk(c, p, drain):
                g = wid * n_chunks + c
                t = g // bg
                bgi = g % bg
                pltpu.sync_copy(
                    idx_hbm.at[t0 + t, pl.ds(1, nf),
                               pl.ds(bgi * LANES, LANES)],
                    idx_v.at[p, pl.ds(0, nf)])
                for fi in range(nf):
                    for j8 in range(LANES // 16):
                        sl = (p, fi, pl.ds(j8 * 16, 16))
                        idx_v[sl] = idx_v[sl] + jnp.int32((1 + fi) * V)
                cps = [
                    pltpu.async_copy(
                        tab_hbm.at[idx_v.at[p, fi]],
                        rows_v.at[p, pl.ds(fi * LANES, LANES)],
                        gsem,
                    )
                    for fi in range(nf)
                ]
                for cp in cps:
                    cp.wait()

                def drain_prev():
                    for fi in range(nf):
                        pltpu.make_async_copy(
                            rows_v.at[p, pl.ds(fi * LANES, LANES)],
                            out_hbm.at[t, fi, pl.ds(bgi * LANES, LANES), :],
                            osems[p],
                        ).wait()
                pl.when(drain)(drain_prev)
                for fi in range(nf):
                    pltpu.async_copy(
                        rows_v.at[p, pl.ds(fi * LANES, LANES)],
                        out_hbm.at[t, fi, pl.ds(bgi * LANES, LANES), :],
                        osems[p],
                    )

            def pair_body(kk, carry):
                do_chunk(2 * kk, 0, kk > 0)
                do_chunk(2 * kk + 1, 1, kk > 0)
                return carry

            lax.fori_loop(0, n_chunks // 2, pair_body, 0)
            for p in range(2):
                for fi in range(nf):
                    pltpu.make_async_copy(
                        rows_v.at[p, pl.ds(fi * LANES, LANES)],
                        out_hbm.at[0, fi, pl.ds(0, LANES), :],
                        osems[p],
                    ).wait()

        phase(oh_hbm, hc, NF_HIST, 0)
        phase(of_hbm, fc, NF_FUT, HIST)

        pltpu.sync_copy(idx_hbm.at[0, 0, pl.ds(wid * s_per_w, s_per_w)], sidx_v)
        pltpu.async_copy(tab_hbm.at[sidx_v], srows_v, gsem).wait()
        pltpu.sync_copy(srows_v, os_hbm.at[pl.ds(wid * s_per_w, s_per_w)])

    return k(tab, idx3)


def kernel(inputs, tables):
    B, W, NI = inputs.shape
    total_window = HIST + PRED
    if W > total_window:
        inputs = inputs[:, -total_window:, :]
        W = total_window
    inputs = inputs.astype(jnp.int32)
    tab = tables.reshape(NI * V, D)

    idx3 = inputs.transpose(1, 2, 0)

    info = plsc.get_sparse_core_info()
    nw = info.num_cores * info.num_subcores

    oh, of, os_ = _gather_all(tab, idx3, B, nw)
    static = os_.reshape(B, 1, D)
    historical = oh.transpose(2, 0, 1, 3)
    future = of.transpose(2, 0, 1, 3)
    return (static, historical, future)
